# baseline (speedup 1.0000x reference)
import functools

import jax
import jax.numpy as jnp
from jax import lax
from jax.experimental import pallas as pl
from jax.experimental.pallas import tpu as pltpu
from jax.experimental.pallas import tpu_sc as plsc

_NC = 2
_NS = 16
_NW = _NC * _NS
_L = 16
_LANE = 128
_NR = 6
_STG = 64
_CNTN = 272


def _body(B, N, D, tableT_hbm, idx_hbm, out_hbm, idxall_v, hiti_v, hitb_v,
          sortb_v, cnt2_v, ofa_v, slabs_v, stage_v, sems):
    *slabsems, outsem = sems
    wid = lax.axis_index("s") * _NC + lax.axis_index("c")
    ntile = (N + _LANE - 1) // _LANE
    tr = (ntile + _NW - 1) // _NW
    tr_pad = ((tr + _NR - 1) // _NR) * _NR
    lo = wid * tr
    iota16 = lax.iota(jnp.int32, _L)
    lane0 = iota16 == 0
    sorti_v = idxall_v

    pltpu.sync_copy(idx_hbm, idxall_v.at[pl.ds(0, B)])

    def c_step(g, cnt):
        v16 = idxall_v[pl.ds(g * _L, _L)]
        t16 = lax.shift_right_logical(v16, 7)
        m = (t16 >= lo) & (t16 < lo + tr)
        pos16 = cnt + plsc.cumsum(m.astype(jnp.int32)) - 1
        plsc.store_scatter(hiti_v, [pos16], v16, mask=m)
        plsc.store_scatter(hitb_v, [pos16], g * _L + iota16, mask=m)
        return cnt + plsc.all_reduce_population_count(m)[0]

    cnt = lax.fori_loop(0, B // _L, c_step, jnp.int32(0))

    for q in range(_CNTN // _L):
        cnt2_v[pl.ds(q * _L, _L)] = jnp.zeros((_L,), jnp.int32)

    def hist_step(h, carry):
        t_rel = lax.shift_right_logical(hiti_v[pl.ds(h, _L)][0], 7) - lo
        c0 = cnt2_v[pl.ds(t_rel, _L)][0]
        plsc.store_scatter(cnt2_v, [jnp.full((_L,), t_rel, jnp.int32)],
                           jnp.full((_L,), c0 + 1, jnp.int32), mask=lane0)
        return carry

    lax.fori_loop(0, cnt, hist_step, 0)

    carry = jnp.int32(0)
    for q in range(_CNTN // _L):
        v = cnt2_v[pl.ds(q * _L, _L)]
        cs = plsc.cumsum(v)
        ofa_v[pl.ds(q * _L, _L)] = cs - v + carry
        carry = carry + cs[_L - 1]

    def place_step(h, carry):
        i = hiti_v[pl.ds(h, _L)][0]
        b = hitb_v[pl.ds(h, _L)][0]
        t_rel = lax.shift_right_logical(i, 7) - lo
        pos = ofa_v[pl.ds(t_rel, _L)][0]
        p16 = jnp.full((_L,), pos, jnp.int32)
        plsc.store_scatter(sorti_v, [p16], jnp.full((_L,), i, jnp.int32),
                           mask=lane0)
        plsc.store_scatter(sortb_v, [p16], jnp.full((_L,), b, jnp.int32),
                           mask=lane0)
        plsc.store_scatter(ofa_v, [jnp.full((_L,), t_rel, jnp.int32)],
                           jnp.full((_L,), pos + 1, jnp.int32), mask=lane0)
        return carry

    lax.fori_loop(0, cnt, place_step, 0)

    def drain_rows(n):
        def one(i, c):
            pltpu.make_async_copy(
                out_hbm.at[jnp.int32(0)], stage_v.at[jnp.int32(0)], outsem
            ).wait()
            return c

        lax.fori_loop(0, n, one, 0)

    def fetch(p, t):
        tt = jnp.minimum(t, jnp.int32(ntile - 1))
        off = pl.multiple_of(tt * _LANE, _LANE)
        pltpu.make_async_copy(
            tableT_hbm.at[:, pl.ds(off, _LANE)], slabs_v.at[p], slabsems[p]
        ).start()

    def slab_wait(p):
        pltpu.make_async_copy(
            tableT_hbm.at[:, pl.ds(0, _LANE)], slabs_v.at[p], slabsems[p]
        ).wait()

    for p in range(_NR):
        fetch(p, jnp.int32(lo + p))

    def process_slab(p, t_rel, carry):
        issued, drained = carry
        end = ofa_v[pl.ds(t_rel, _L)][0]
        n = cnt2_v[pl.ds(t_rel, _L)][0]
        start = end - n

        def h_step(q, c2):
            iss, drn = c2
            k16 = q * _L + iota16
            m = k16 < n
            hv16 = sorti_v[pl.ds(start + q * _L, _L)]
            hb16 = sortb_v[pl.ds(start + q * _L, _L)]
            npc = plsc.all_reduce_population_count(m)[0]
            waitrows = jnp.maximum(iss - drn - jnp.int32(_STG - _L), 0)
            drain_rows(waitrows)
            drn = drn + waitrows

            rlo16 = lax.bitwise_and(hv16, jnp.full((_L,), 127, jnp.int32))
            pos16 = lax.rem(iss + plsc.cumsum(m.astype(jnp.int32)) - 1,
                            jnp.int32(_STG))
            for w in range(D):
                vals = plsc.load_gather(
                    slabs_v.at[p], [jnp.full((_L,), w, jnp.int32), rlo16]
                )
                plsc.store_scatter(
                    stage_v, [pos16, jnp.full((_L,), w, jnp.int32)],
                    vals, mask=m,
                )
            mi16 = m.astype(jnp.int32)
            for l in range(_L):
                @pl.when(mi16[l] != 0)
                def _write(l=l):
                    pltpu.make_async_copy(
                        stage_v.at[pos16[l]], out_hbm.at[hb16[l]], outsem
                    ).start()

            return iss + npc, drn

        return lax.fori_loop(0, (n + _L - 1) // _L, h_step,
                             (issued, drained))

    def g_step(G, carry):
        for p in range(_NR):
            t_rel = G * _NR + p
            slab_wait(p)
            carry = process_slab(p, t_rel, carry)
            fetch(p, lo + t_rel + _NR)
        return carry

    issued, drained = lax.fori_loop(0, tr_pad // _NR, g_step,
                                    (jnp.int32(0), jnp.int32(0)))
    for p in range(_NR):
        slab_wait(p)
    drain_rows(issued - drained)


@functools.cache
def _build(B, N, D):
    mesh = plsc.VectorSubcoreMesh(core_axis_name="c", subcore_axis_name="s")
    return pl.kernel(
        functools.partial(_body, B, N, D),
        mesh=mesh,
        out_type=jax.ShapeDtypeStruct((B, D), jnp.float32),
        scratch_types=[
            pltpu.VMEM((B + _L,), jnp.int32),
            pltpu.VMEM((B + _L,), jnp.int32),
            pltpu.VMEM((B + _L,), jnp.int32),
            pltpu.VMEM((B + _L,), jnp.int32),
            pltpu.VMEM((_CNTN,), jnp.int32),
            pltpu.VMEM((_CNTN,), jnp.int32),
            pltpu.VMEM((_NR, D, _LANE), jnp.float32),
            pltpu.VMEM((_STG, D), jnp.float32),
            [pltpu.SemaphoreType.DMA] * (_NR + 1),
        ],
        compiler_params=pltpu.CompilerParams(needs_layout_passes=False),
    )


def kernel(idx, emb_weight):
    B = idx.shape[0]
    N, D = emb_weight.shape
    return _build(B, N, D)(emb_weight.T, idx.astype(jnp.int32))


# speedup vs baseline: 1.3698x; 1.3698x over previous
import functools

import jax
import jax.numpy as jnp
from jax import lax
from jax.experimental import pallas as pl
from jax.experimental.pallas import tpu as pltpu
from jax.experimental.pallas import tpu_sc as plsc

_NC = 2
_NS = 16
_NW = _NC * _NS
_L = 16
_LANE = 128
_NR = 8


def _extract(slabs_v, p, rlo, rows_v, jrow, D):
    rlo16 = jnp.full((_L,), rlo, jnp.int32)
    for k in range(D // _L):
        c16 = lax.iota(jnp.int32, _L) + k * _L
        v = plsc.load_gather(slabs_v.at[p], [c16, rlo16])
        plsc.store_scatter(
            rows_v, [jnp.full((_L,), jrow, jnp.int32), c16], v
        )


def _body(b_per_w, D, tableT_hbm, idx_hbm, out_hbm, idx_v, rows_v, slabs_v,
          *sems):
    wid = lax.axis_index("s") * _NC + lax.axis_index("c")
    base = wid * b_per_w
    ngrp = b_per_w // _L
    pltpu.sync_copy(idx_hbm.at[pl.ds(base, b_per_w)], idx_v)

    def fetch(p, i):
        off = pl.multiple_of((i // _LANE) * _LANE, _LANE)
        pltpu.make_async_copy(
            tableT_hbm.at[:, pl.ds(off, _LANE)], slabs_v.at[p], sems[p]
        ).start()

    def slab_wait(p):
        pltpu.make_async_copy(
            tableT_hbm.at[:, pl.ds(0, _LANE)], slabs_v.at[p], sems[p]
        ).wait()

    v0 = idx_v[pl.ds(0, _L)]
    for p in range(_NR):
        fetch(p, v0[p])

    nflush = rows_v.shape[0]

    def g_step(g, carry):
        v16 = idx_v[pl.ds(g * _L, _L)]
        gn = lax.rem(g + 1, jnp.int32(ngrp))
        v16n = idx_v[pl.ds(gn * _L, _L)]
        jrow = lax.rem(g, jnp.int32(nflush // _L)) * _L
        for l in range(_L):
            p = l % _NR
            slab_wait(p)
            _extract(slabs_v, p, v16[l] % _LANE, rows_v, jrow + l, D)
            nxt = v16[l + _NR] if l + _NR < _L else v16n[l + _NR - _L]
            fetch(p, nxt)

        @pl.when(lax.rem(g, jnp.int32(nflush // _L)) == nflush // _L - 1)
        def _flush():
            pltpu.sync_copy(
                rows_v,
                out_hbm.at[pl.ds(base + (g - (nflush // _L - 1)) * _L, nflush)],
            )

        return carry

    lax.fori_loop(0, ngrp, g_step, None)
    for p in range(_NR):
        slab_wait(p)


@functools.cache
def _build(B, N, D):
    assert B % (8 * _NW) == 0 and D % _L == 0
    b_per_w = B // _NW
    mesh = plsc.VectorSubcoreMesh(core_axis_name="c", subcore_axis_name="s")
    return pl.kernel(
        functools.partial(_body, b_per_w, D),
        mesh=mesh,
        out_type=jax.ShapeDtypeStruct((B, D), jnp.float32),
        scratch_types=[
            pltpu.VMEM((b_per_w,), jnp.int32),
            pltpu.VMEM((128, D), jnp.float32),
            pltpu.VMEM((_NR, D, _LANE), jnp.float32),
        ] + [pltpu.SemaphoreType.DMA] * _NR,
        compiler_params=pltpu.CompilerParams(needs_layout_passes=False),
    )


def kernel(idx, emb_weight):
    B = idx.shape[0]
    N, D = emb_weight.shape
    return _build(B, N, D)(emb_weight.T, idx.astype(jnp.int32))
